# baseline (speedup 1.0000x reference)
import functools

import jax
import jax.numpy as jnp
from jax import lax
from jax.experimental import pallas as pl
from jax.experimental.pallas import tpu as pltpu
from jax.experimental.pallas import tpu_sc as plsc

B, L, V, D = 4096, 200, 100000, 128
LP = 208
LANES = 16
NC, NS = 2, 16
NW = NC * NS
BPW = B // NW
G0, G1 = 128, LP - 128


def _sc_sums(emb, idxm):
    mesh = plsc.VectorSubcoreMesh(core_axis_name="c", subcore_axis_name="s")

    @functools.partial(
        pl.kernel,
        out_type=jax.ShapeDtypeStruct((B, D), jnp.float32),
        mesh=mesh,
        scratch_types=[
            pltpu.VMEM((G0,), jnp.int32),
            pltpu.VMEM((G1,), jnp.int32),
            pltpu.VMEM((LP, D), jnp.float32),
            pltpu.VMEM((BPW, D), jnp.float32),
            pltpu.SemaphoreType.DMA,
        ],
    )
    def k(emb_hbm, idx_hbm, out_hbm, idx0_v, idx1_v, rows_v, out_v, sem):
        wid = lax.axis_index("s") * NC + lax.axis_index("c")
        base = wid * BPW

        def body(b, carry):
            pltpu.sync_copy(idx_hbm.at[base + b, pl.ds(0, G0)], idx0_v)
            pltpu.sync_copy(idx_hbm.at[base + b, pl.ds(G0, G1)], idx1_v)
            cp0 = pltpu.async_copy(emb_hbm.at[idx0_v], rows_v.at[pl.ds(0, G0)], sem)
            cp1 = pltpu.async_copy(emb_hbm.at[idx1_v], rows_v.at[pl.ds(G0, G1)], sem)
            cp0.wait()
            cp1.wait()

            def row(j, acc):
                return tuple(acc[c] + rows_v[j, pl.ds(c * LANES, LANES)]
                             for c in range(D // LANES))

            acc = lax.fori_loop(
                0, LP, row,
                tuple(jnp.zeros((LANES,), jnp.float32) for _ in range(D // LANES)))
            for c in range(D // LANES):
                out_v[b, pl.ds(c * LANES, LANES)] = acc[c]
            return carry

        lax.fori_loop(0, BPW, body, 0)
        pltpu.sync_copy(out_v, out_hbm.at[pl.ds(base, BPW)])

    return k(emb, idxm)


def _tc_finish(sums, mask, proj_t, emb0):
    BB = 512

    def body(s_ref, m_ref, p_ref, e0_ref, o_ref):
        cnt = jnp.sum(m_ref[...].astype(jnp.float32), axis=1, keepdims=True)
        corr = s_ref[...] - (LP - cnt) * e0_ref[...]
        y = jnp.dot(corr, p_ref[...], preferred_element_type=jnp.float32)
        o_ref[...] = y / jnp.clip(cnt, 1e-9, None)

    return pl.pallas_call(
        body,
        grid=(B // BB,),
        in_specs=[
            pl.BlockSpec((BB, D), lambda i: (i, 0)),
            pl.BlockSpec((BB, L), lambda i: (i, 0)),
            pl.BlockSpec((D, D), lambda i: (0, 0)),
            pl.BlockSpec((1, D), lambda i: (0, 0)),
        ],
        out_specs=pl.BlockSpec((BB, D), lambda i: (i, 0)),
        out_shape=jax.ShapeDtypeStruct((B, D), jnp.float32),
    )(sums, mask, proj_t, emb0)


def kernel(input_ids, attention_mask, emb, proj):
    ids = input_ids.astype(jnp.int32)
    msk = attention_mask.astype(jnp.int32)
    idxm = jnp.pad(ids * msk, ((0, 0), (0, LP - L)))
    sums = _sc_sums(emb, idxm)
    return _tc_finish(sums, msk, proj.T, emb[0:1])
